# baseline (speedup 1.0000x reference)
import functools

import jax
import jax.numpy as jnp
from jax import lax
from jax.experimental import pallas as pl
from jax.experimental.pallas import tpu as pltpu
from jax.experimental.pallas import tpu_sc as plsc

N = 10000
E = 320000
D = 128
C = 41

NC = 2
NS = 16
NW = NC * NS
EPW = E // NW
K = 80
CHUNKS = EPW // K
RPT = 624
TAIL = N - NS * RPT

_GATHER_1D = lax.GatherDimensionNumbers(
    offset_dims=(), collapsed_slice_dims=(0,), start_index_map=(0,))


def _spmm_sc_body(g_hbm, src_hbm, dst_hbm, vals_hbm, zeros_hbm,
                  out0_hbm, out1_hbm,
                  src_v0, src_v1, src_v2, src_v3,
                  dst_v0, dst_v1, dst_v2, dst_v3,
                  val_v0, val_v1, val_v2, val_v3,
                  rows_v0, rows_v1, rows_v2, rows_v3,
                  acc,
                  sg0, sg1, sg2, sg3,
                  sh0, sh1, sh2, sh3,
                  ss0, ss1, ss2, ss3,
                  sd0, sd1, sd2, sd3, sem_z):
    c = lax.axis_index("c")
    s = lax.axis_index("s")
    wid = s * NC + c

    srcv = [src_v0, src_v1, src_v2, src_v3]
    dstv = [dst_v0, dst_v1, dst_v2, dst_v3]
    valv = [val_v0, val_v1, val_v2, val_v3]
    rows = [rows_v0, rows_v1, rows_v2, rows_v3]
    sem_g = [sg0, sg1, sg2, sg3]
    sem_g2 = [sh0, sh1, sh2, sh3]
    sem_s = [ss0, ss1, ss2, ss3]
    sem_d = [sd0, sd1, sd2, sd3]

    rbase = s * RPT
    pltpu.async_copy(zeros_hbm.at[pl.ds(rbase, RPT)],
                     acc.at[pl.ds(rbase, RPT)], sem_z)

    @pl.when(s == NS - 1)
    def _zero_tail():
        pltpu.async_copy(zeros_hbm.at[pl.ds(NS * RPT, TAIL)],
                         acc.at[pl.ds(NS * RPT, TAIL)], sem_z)

    ebase = wid * EPW

    def start_idx(i, b):
        base = ebase + i * K
        pltpu.async_copy(src_hbm.at[pl.ds(base, K)], srcv[b], sem_d[b])
        pltpu.async_copy(dst_hbm.at[pl.ds(base, K)], dstv[b], sem_d[b])
        pltpu.async_copy(vals_hbm.at[pl.ds(base, K)], valv[b], sem_d[b])

    def wait_idx(b):
        pltpu.make_async_copy(src_hbm.at[pl.ds(0, K)], srcv[b], sem_d[b]).wait()
        pltpu.make_async_copy(dst_hbm.at[pl.ds(0, K)], dstv[b], sem_d[b]).wait()
        pltpu.make_async_copy(vals_hbm.at[pl.ds(0, K)], valv[b], sem_d[b]).wait()

    H = K // 2

    def start_gather(b):
        pltpu.async_copy(g_hbm.at[srcv[b].at[pl.ds(0, H)]],
                         rows[b].at[pl.ds(0, H)], sem_g[b])
        pltpu.async_copy(g_hbm.at[srcv[b].at[pl.ds(H, H)]],
                         rows[b].at[pl.ds(H, H)], sem_g2[b])

    def wait_gather(b):
        pltpu.make_async_copy(g_hbm.at[srcv[b].at[pl.ds(0, H)]],
                              rows[b].at[pl.ds(0, H)], sem_g[b]).wait()
        pltpu.make_async_copy(g_hbm.at[srcv[b].at[pl.ds(H, H)]],
                              rows[b].at[pl.ds(H, H)], sem_g2[b]).wait()

    def start_scatter(b):
        pltpu.async_copy(rows[b], acc.at[dstv[b]], sem_s[b], add=True)

    def wait_scatter(b):
        pltpu.make_async_copy(rows[b], acc.at[dstv[b]], sem_s[b]).wait()

    def mul(b):
        rref = rows[b]
        vref = valv[b]

        def mul_body(g2, carry2):
            val16 = vref[pl.ds(g2 * 16, 16)]
            for j in range(16):
                v = lax.gather(
                    val16, jnp.full((16, 1), j, jnp.int32), _GATHER_1D,
                    slice_sizes=(1,),
                    mode=lax.GatherScatterMode.PROMISE_IN_BOUNDS)
                e = g2 * 16 + j
                for dd in range(D // 16):
                    sl = pl.ds(dd * 16, 16)
                    rref[e, sl] = rref[e, sl] * v
            return carry2

        lax.fori_loop(0, K // 16, mul_body, 0)

    start_idx(0, 0); start_idx(1, 1); start_idx(2, 2); start_idx(3, 3)
    wait_idx(0); start_gather(0)
    wait_idx(1); start_gather(1)
    pltpu.make_async_copy(zeros_hbm.at[pl.ds(rbase, RPT)],
                          acc.at[pl.ds(rbase, RPT)], sem_z).wait()

    @pl.when(s == NS - 1)
    def _zero_tail_wait():
        pltpu.make_async_copy(zeros_hbm.at[pl.ds(NS * RPT, TAIL)],
                              acc.at[pl.ds(NS * RPT, TAIL)], sem_z).wait()

    plsc.subcore_barrier()
    wait_gather(0); mul(0); start_scatter(0)
    wait_idx(2); start_gather(2)
    wait_gather(1); mul(1); start_scatter(1)
    wait_idx(3); start_gather(3)
    wait_scatter(0); start_idx(4, 0)

    def loop_body(ih, carry):
        i0 = 2 + ih * 4
        for t in range(4):
            i = i0 + t
            b = (2 + t) % 4
            wait_gather(b); mul(b); start_scatter(b)
            b2 = (b + 2) % 4
            wait_idx(b2); start_gather(b2)
            b3 = (b + 3) % 4
            wait_scatter(b3)
            start_idx(i + 3, b3)
        return carry

    lax.fori_loop(0, (CHUNKS - 5) // 4, loop_body, 0)

    wait_gather(2); mul(2); start_scatter(2)
    wait_idx(0); start_gather(0)
    wait_scatter(1)
    wait_gather(3); mul(3); start_scatter(3)
    wait_scatter(2)
    wait_gather(0); mul(0); start_scatter(0)
    wait_scatter(3)
    wait_scatter(0)
    plsc.subcore_barrier()

    @pl.when(c == 0)
    def _out_c0():
        pltpu.sync_copy(acc.at[pl.ds(rbase, RPT)],
                        out0_hbm.at[pl.ds(rbase, RPT)])

        @pl.when(s == NS - 1)
        def _out_tail0():
            pltpu.sync_copy(acc.at[pl.ds(NS * RPT, TAIL)],
                            out0_hbm.at[pl.ds(NS * RPT, TAIL)])

    @pl.when(c == 1)
    def _out_c1():
        pltpu.sync_copy(acc.at[pl.ds(rbase, RPT)],
                        out1_hbm.at[pl.ds(rbase, RPT)])

        @pl.when(s == NS - 1)
        def _out_tail1():
            pltpu.sync_copy(acc.at[pl.ds(NS * RPT, TAIL)],
                            out1_hbm.at[pl.ds(NS * RPT, TAIL)])


def _make_spmm():
    mesh = plsc.VectorSubcoreMesh(core_axis_name="c", subcore_axis_name="s")
    return functools.partial(
        pl.kernel,
        mesh=mesh,
        out_type=[jax.ShapeDtypeStruct((N, D), jnp.float32),
                  jax.ShapeDtypeStruct((N, D), jnp.float32)],
        scratch_types=(
            [pltpu.VMEM((K,), jnp.int32)] * 4
            + [pltpu.VMEM((K,), jnp.int32)] * 4
            + [pltpu.VMEM((K,), jnp.float32)] * 4
            + [pltpu.VMEM((K, D), jnp.float32)] * 4
            + [pltpu.VMEM_SHARED((N, D), jnp.float32)]
            + [pltpu.SemaphoreType.DMA] * 17
        ),
    )(_spmm_sc_body)


BN = 2000
GRID = N // BN


def _ln(f, sc, of):
    mean = jnp.mean(f, axis=1, keepdims=True)
    cent = f - mean
    var = jnp.mean(cent * cent, axis=1, keepdims=True) + 1e-9
    return cent * sc * lax.rsqrt(var) + of


def _dot(a, b):
    return jnp.dot(a, b, preferred_element_type=jnp.float32)


def _tc1g_body(feat, W01, g0_o):
    g0_o[...] = _dot(feat[...], W01[...])


def _tc1f_body(feat, W00, b00, sc00, of00, f0_o):
    h = _dot(feat[...], W00[...]) + b00[...]
    f0_o[...] = _ln(jnp.maximum(h, 0.0), sc00[...], of00[...])


def _tc2g_body(p0, p1, b01, sc01, of01, f0, W11a, W11b, g1_o, f1_o):
    s0 = p0[...] + p1[...]
    f1 = _ln(jnp.maximum(s0 + b01[...], 0.0), sc01[...], of01[...])
    f1_o[...] = f1
    g1_o[...] = _dot(f0[...], W11a[...]) + _dot(f1, W11b[...])


def _tc2f_body(f0, f1, W10a, W10b, b10, sc10, of10, f0n_o):
    h = _dot(f0[...], W10a[...]) + _dot(f1[...], W10b[...]) + b10[...]
    f0n_o[...] = _ln(jnp.maximum(h, 0.0), sc10[...], of10[...])


def _tc3_body(p0, p1, b11, sc11, of11, f0n, W20a, W20b, b20, sc20, of20,
              Wc, bc, out_o):
    s1 = p0[...] + p1[...]
    f1 = _ln(jnp.maximum(s1 + b11[...], 0.0), sc11[...], of11[...])
    h = _dot(f0n[...], W20a[...]) + _dot(f1, W20b[...]) + b20[...]
    f2 = _ln(jnp.maximum(h, 0.0), sc20[...], of20[...])
    nrm = jnp.sqrt(jnp.sum(f2 * f2, axis=1, keepdims=True))
    emb = f2 / jnp.maximum(nrm, 1e-12)
    out_o[...] = _dot(emb, Wc[...]) + bc[...]


def _row_spec():
    return pl.BlockSpec((BN, D), lambda i: (i, 0))


def _full_spec(shape):
    return pl.BlockSpec(shape, lambda i: (0,) * len(shape))


def _tc1g(feat, W01):
    return pl.pallas_call(
        _tc1g_body,
        grid=(GRID,),
        in_specs=[_row_spec(), _full_spec((D, D))],
        out_specs=[_row_spec()],
        out_shape=[jax.ShapeDtypeStruct((N, D), jnp.float32)],
    )(feat, W01)[0]


def _tc1f(feat, W00, b00, sc00, of00):
    return pl.pallas_call(
        _tc1f_body,
        grid=(GRID,),
        in_specs=[_row_spec(), _full_spec((D, D)), _full_spec((1, D)),
                  _full_spec((1, D)), _full_spec((1, D))],
        out_specs=[_row_spec()],
        out_shape=[jax.ShapeDtypeStruct((N, D), jnp.float32)],
    )(feat, W00, b00, sc00, of00)[0]


def _tc2g(p0, p1, b01, sc01, of01, f0, W11a, W11b):
    return pl.pallas_call(
        _tc2g_body,
        grid=(GRID,),
        in_specs=[_row_spec(), _row_spec(), _full_spec((1, D)),
                  _full_spec((1, D)), _full_spec((1, D)), _row_spec(),
                  _full_spec((D, D)), _full_spec((D, D))],
        out_specs=[_row_spec(), _row_spec()],
        out_shape=[jax.ShapeDtypeStruct((N, D), jnp.float32),
                   jax.ShapeDtypeStruct((N, D), jnp.float32)],
    )(p0, p1, b01, sc01, of01, f0, W11a, W11b)


def _tc2f(f0, f1, W10a, W10b, b10, sc10, of10):
    return pl.pallas_call(
        _tc2f_body,
        grid=(GRID,),
        in_specs=[_row_spec(), _row_spec(), _full_spec((D, D)),
                  _full_spec((D, D)), _full_spec((1, D)),
                  _full_spec((1, D)), _full_spec((1, D))],
        out_specs=[_row_spec()],
        out_shape=[jax.ShapeDtypeStruct((N, D), jnp.float32)],
    )(f0, f1, W10a, W10b, b10, sc10, of10)[0]


def _tc3(p0, p1, b11, sc11, of11, f0n, W20a, W20b, b20, sc20, of20, Wc, bc):
    return pl.pallas_call(
        _tc3_body,
        grid=(GRID,),
        in_specs=[_row_spec(), _row_spec(), _full_spec((1, D)),
                  _full_spec((1, D)), _full_spec((1, D)), _row_spec(),
                  _full_spec((D, D)), _full_spec((D, D)), _full_spec((1, D)),
                  _full_spec((1, D)), _full_spec((1, D)),
                  _full_spec((D, C)), _full_spec((1, C))],
        out_specs=[pl.BlockSpec((BN, C), lambda i: (i, 0))],
        out_shape=[jax.ShapeDtypeStruct((N, C), jnp.float32)],
    )(p0, p1, b11, sc11, of11, f0n, W20a, W20b, b20, sc20, of20, Wc, bc)


def kernel(feat_subg, edge_index, edge_vals,
           W_0_0, b_0_0, sc_0_0, of_0_0,
           W_0_1, b_0_1, sc_0_1, of_0_1,
           W_1_0, b_1_0, sc_1_0, of_1_0,
           W_1_1, b_1_1, sc_1_1, of_1_1,
           W_2_0, b_2_0, sc_2_0, of_2_0,
           W_cls, b_cls):
    spmm = _make_spmm()

    src = edge_index[1]
    dst = edge_index[0]
    zeros = jnp.zeros((N, D), jnp.float32)

    r = lambda v: v.reshape(1, -1)

    g0 = _tc1g(feat_subg, W_0_1)

    p0a, p0b = spmm(g0, src, dst, edge_vals, zeros)

    f0 = _tc1f(feat_subg, W_0_0, r(b_0_0), r(sc_0_0), r(of_0_0))

    g1, f1 = _tc2g(p0a, p0b, r(b_0_1), r(sc_0_1), r(of_0_1), f0,
                   W_1_1[:D], W_1_1[D:])

    p1a, p1b = spmm(g1, src, dst, edge_vals, zeros)

    f0n = _tc2f(f0, f1, W_1_0[:D], W_1_0[D:], r(b_1_0), r(sc_1_0),
                r(of_1_0))

    (out,) = _tc3(p1a, p1b, r(b_1_1), r(sc_1_1), r(of_1_1), f0n,
                  W_2_0[:D], W_2_0[D:], r(b_2_0), r(sc_2_0), r(of_2_0),
                  W_cls, r(b_cls))
    return out


# speedup vs baseline: 1.0106x; 1.0106x over previous
import functools

import jax
import jax.numpy as jnp
from jax import lax
from jax.experimental import pallas as pl
from jax.experimental.pallas import tpu as pltpu
from jax.experimental.pallas import tpu_sc as plsc

N = 10000
E = 320000
D = 128
C = 41

NC = 2
NS = 16
NW = NC * NS
EPW = E // NW
K = 80
CHUNKS = EPW // K
RPT = 624
TAIL = N - NS * RPT

_GATHER_1D = lax.GatherDimensionNumbers(
    offset_dims=(), collapsed_slice_dims=(0,), start_index_map=(0,))


def _spmm_sc_body(g_hbm, src_hbm, dst_hbm, vals_hbm, zeros_hbm,
                  out0_hbm, out1_hbm,
                  src_v0, src_v1, src_v2, src_v3,
                  dst_v0, dst_v1, dst_v2, dst_v3,
                  val_v0, val_v1, val_v2, val_v3,
                  rows_v0, rows_v1, rows_v2, rows_v3,
                  acc,
                  sg0, sg1, sg2, sg3,
                  ss0, ss1, ss2, ss3,
                  sd0, sd1, sd2, sd3, sem_z):
    c = lax.axis_index("c")
    s = lax.axis_index("s")
    wid = s * NC + c

    srcv = [src_v0, src_v1, src_v2, src_v3]
    dstv = [dst_v0, dst_v1, dst_v2, dst_v3]
    valv = [val_v0, val_v1, val_v2, val_v3]
    rows = [rows_v0, rows_v1, rows_v2, rows_v3]
    sem_g = [sg0, sg1, sg2, sg3]
    sem_s = [ss0, ss1, ss2, ss3]
    sem_d = [sd0, sd1, sd2, sd3]

    rbase = s * RPT
    pltpu.async_copy(zeros_hbm.at[pl.ds(rbase, RPT)],
                     acc.at[pl.ds(rbase, RPT)], sem_z)

    @pl.when(s == NS - 1)
    def _zero_tail():
        pltpu.async_copy(zeros_hbm.at[pl.ds(NS * RPT, TAIL)],
                         acc.at[pl.ds(NS * RPT, TAIL)], sem_z)

    ebase = wid * EPW

    def start_idx(i, b):
        base = ebase + i * K
        pltpu.async_copy(src_hbm.at[pl.ds(base, K)], srcv[b], sem_d[b])
        pltpu.async_copy(dst_hbm.at[pl.ds(base, K)], dstv[b], sem_d[b])
        pltpu.async_copy(vals_hbm.at[pl.ds(base, K)], valv[b], sem_d[b])

    def wait_idx(b):
        pltpu.make_async_copy(src_hbm.at[pl.ds(0, K)], srcv[b], sem_d[b]).wait()
        pltpu.make_async_copy(dst_hbm.at[pl.ds(0, K)], dstv[b], sem_d[b]).wait()
        pltpu.make_async_copy(vals_hbm.at[pl.ds(0, K)], valv[b], sem_d[b]).wait()

    def start_gather(b):
        pltpu.async_copy(g_hbm.at[srcv[b]], rows[b], sem_g[b])

    def wait_gather(b):
        pltpu.make_async_copy(g_hbm.at[srcv[b]], rows[b], sem_g[b]).wait()

    def start_scatter(b):
        pltpu.async_copy(rows[b], acc.at[dstv[b]], sem_s[b], add=True)

    def wait_scatter(b):
        pltpu.make_async_copy(rows[b], acc.at[dstv[b]], sem_s[b]).wait()

    def mul(b):
        rref = rows[b]
        vref = valv[b]

        def mul_body(g2, carry2):
            val16 = vref[pl.ds(g2 * 16, 16)]
            for j in range(16):
                v = lax.gather(
                    val16, jnp.full((16, 1), j, jnp.int32), _GATHER_1D,
                    slice_sizes=(1,),
                    mode=lax.GatherScatterMode.PROMISE_IN_BOUNDS)
                e = g2 * 16 + j
                for dd in range(D // 16):
                    sl = pl.ds(dd * 16, 16)
                    rref[e, sl] = rref[e, sl] * v
            return carry2

        lax.fori_loop(0, K // 16, mul_body, 0)

    start_idx(0, 0); start_idx(1, 1); start_idx(2, 2); start_idx(3, 3)
    wait_idx(0); start_gather(0)
    wait_idx(1); start_gather(1)
    pltpu.make_async_copy(zeros_hbm.at[pl.ds(rbase, RPT)],
                          acc.at[pl.ds(rbase, RPT)], sem_z).wait()

    @pl.when(s == NS - 1)
    def _zero_tail_wait():
        pltpu.make_async_copy(zeros_hbm.at[pl.ds(NS * RPT, TAIL)],
                              acc.at[pl.ds(NS * RPT, TAIL)], sem_z).wait()

    plsc.subcore_barrier()
    wait_gather(0); mul(0); start_scatter(0)
    wait_idx(2); start_gather(2)
    wait_gather(1); mul(1); start_scatter(1)
    wait_idx(3); start_gather(3)
    wait_scatter(0); start_idx(4, 0)

    def loop_body(ih, carry):
        i0 = 2 + ih * 4
        for t in range(4):
            i = i0 + t
            b = (2 + t) % 4
            wait_gather(b); mul(b); start_scatter(b)
            b2 = (b + 2) % 4
            wait_idx(b2); start_gather(b2)
            b3 = (b + 3) % 4
            wait_scatter(b3)
            start_idx(i + 3, b3)
        return carry

    lax.fori_loop(0, (CHUNKS - 5) // 4, loop_body, 0)

    wait_gather(2); mul(2); start_scatter(2)
    wait_idx(0); start_gather(0)
    wait_scatter(1)
    wait_gather(3); mul(3); start_scatter(3)
    wait_scatter(2)
    wait_gather(0); mul(0); start_scatter(0)
    wait_scatter(3)
    wait_scatter(0)
    plsc.subcore_barrier()

    @pl.when(c == 0)
    def _out_c0():
        pltpu.sync_copy(acc.at[pl.ds(rbase, RPT)],
                        out0_hbm.at[pl.ds(rbase, RPT)])

        @pl.when(s == NS - 1)
        def _out_tail0():
            pltpu.sync_copy(acc.at[pl.ds(NS * RPT, TAIL)],
                            out0_hbm.at[pl.ds(NS * RPT, TAIL)])

    @pl.when(c == 1)
    def _out_c1():
        pltpu.sync_copy(acc.at[pl.ds(rbase, RPT)],
                        out1_hbm.at[pl.ds(rbase, RPT)])

        @pl.when(s == NS - 1)
        def _out_tail1():
            pltpu.sync_copy(acc.at[pl.ds(NS * RPT, TAIL)],
                            out1_hbm.at[pl.ds(NS * RPT, TAIL)])


def _make_spmm():
    mesh = plsc.VectorSubcoreMesh(core_axis_name="c", subcore_axis_name="s")
    return functools.partial(
        pl.kernel,
        mesh=mesh,
        out_type=[jax.ShapeDtypeStruct((N, D), jnp.float32),
                  jax.ShapeDtypeStruct((N, D), jnp.float32)],
        scratch_types=(
            [pltpu.VMEM((K,), jnp.int32)] * 4
            + [pltpu.VMEM((K,), jnp.int32)] * 4
            + [pltpu.VMEM((K,), jnp.float32)] * 4
            + [pltpu.VMEM((K, D), jnp.float32)] * 4
            + [pltpu.VMEM_SHARED((N, D), jnp.float32)]
            + [pltpu.SemaphoreType.DMA] * 13
        ),
    )(_spmm_sc_body)


BN = 2000
GRID = N // BN


def _ln(f, sc, of):
    mean = jnp.mean(f, axis=1, keepdims=True)
    cent = f - mean
    var = jnp.mean(cent * cent, axis=1, keepdims=True) + 1e-9
    return cent * sc * lax.rsqrt(var) + of


def _dot(a, b):
    return jnp.dot(a, b, preferred_element_type=jnp.float32)


def _tc1g_body(feat, W01, g0_o):
    g0_o[...] = _dot(feat[...], W01[...])


def _tc1f_body(feat, W00, b00, sc00, of00, f0_o):
    h = _dot(feat[...], W00[...]) + b00[...]
    f0_o[...] = _ln(jnp.maximum(h, 0.0), sc00[...], of00[...])


def _tc2g_body(p0, p1, b01, sc01, of01, f0, W11a, W11b, g1_o, f1_o):
    s0 = p0[...] + p1[...]
    f1 = _ln(jnp.maximum(s0 + b01[...], 0.0), sc01[...], of01[...])
    f1_o[...] = f1
    g1_o[...] = _dot(f0[...], W11a[...]) + _dot(f1, W11b[...])


def _tc2f_body(f0, f1, W10a, W10b, b10, sc10, of10, f0n_o):
    h = _dot(f0[...], W10a[...]) + _dot(f1[...], W10b[...]) + b10[...]
    f0n_o[...] = _ln(jnp.maximum(h, 0.0), sc10[...], of10[...])


def _tc3_body(p0, p1, b11, sc11, of11, f0n, W20a, W20b, b20, sc20, of20,
              Wc, bc, out_o):
    s1 = p0[...] + p1[...]
    f1 = _ln(jnp.maximum(s1 + b11[...], 0.0), sc11[...], of11[...])
    h = _dot(f0n[...], W20a[...]) + _dot(f1, W20b[...]) + b20[...]
    f2 = _ln(jnp.maximum(h, 0.0), sc20[...], of20[...])
    nrm = jnp.sqrt(jnp.sum(f2 * f2, axis=1, keepdims=True))
    emb = f2 / jnp.maximum(nrm, 1e-12)
    out_o[...] = _dot(emb, Wc[...]) + bc[...]


def _row_spec():
    return pl.BlockSpec((BN, D), lambda i: (i, 0))


def _full_spec(shape):
    return pl.BlockSpec(shape, lambda i: (0,) * len(shape))


def _tc1g(feat, W01):
    return pl.pallas_call(
        _tc1g_body,
        grid=(GRID,),
        in_specs=[_row_spec(), _full_spec((D, D))],
        out_specs=[_row_spec()],
        out_shape=[jax.ShapeDtypeStruct((N, D), jnp.float32)],
    )(feat, W01)[0]


def _tc1f(feat, W00, b00, sc00, of00):
    return pl.pallas_call(
        _tc1f_body,
        grid=(GRID,),
        in_specs=[_row_spec(), _full_spec((D, D)), _full_spec((1, D)),
                  _full_spec((1, D)), _full_spec((1, D))],
        out_specs=[_row_spec()],
        out_shape=[jax.ShapeDtypeStruct((N, D), jnp.float32)],
    )(feat, W00, b00, sc00, of00)[0]


def _tc2g(p0, p1, b01, sc01, of01, f0, W11a, W11b):
    return pl.pallas_call(
        _tc2g_body,
        grid=(GRID,),
        in_specs=[_row_spec(), _row_spec(), _full_spec((1, D)),
                  _full_spec((1, D)), _full_spec((1, D)), _row_spec(),
                  _full_spec((D, D)), _full_spec((D, D))],
        out_specs=[_row_spec(), _row_spec()],
        out_shape=[jax.ShapeDtypeStruct((N, D), jnp.float32),
                   jax.ShapeDtypeStruct((N, D), jnp.float32)],
    )(p0, p1, b01, sc01, of01, f0, W11a, W11b)


def _tc2f(f0, f1, W10a, W10b, b10, sc10, of10):
    return pl.pallas_call(
        _tc2f_body,
        grid=(GRID,),
        in_specs=[_row_spec(), _row_spec(), _full_spec((D, D)),
                  _full_spec((D, D)), _full_spec((1, D)),
                  _full_spec((1, D)), _full_spec((1, D))],
        out_specs=[_row_spec()],
        out_shape=[jax.ShapeDtypeStruct((N, D), jnp.float32)],
    )(f0, f1, W10a, W10b, b10, sc10, of10)[0]


def _tc3(p0, p1, b11, sc11, of11, f0n, W20a, W20b, b20, sc20, of20, Wc, bc):
    return pl.pallas_call(
        _tc3_body,
        grid=(GRID,),
        in_specs=[_row_spec(), _row_spec(), _full_spec((1, D)),
                  _full_spec((1, D)), _full_spec((1, D)), _row_spec(),
                  _full_spec((D, D)), _full_spec((D, D)), _full_spec((1, D)),
                  _full_spec((1, D)), _full_spec((1, D)),
                  _full_spec((D, C)), _full_spec((1, C))],
        out_specs=[pl.BlockSpec((BN, C), lambda i: (i, 0))],
        out_shape=[jax.ShapeDtypeStruct((N, C), jnp.float32)],
    )(p0, p1, b11, sc11, of11, f0n, W20a, W20b, b20, sc20, of20, Wc, bc)


def kernel(feat_subg, edge_index, edge_vals,
           W_0_0, b_0_0, sc_0_0, of_0_0,
           W_0_1, b_0_1, sc_0_1, of_0_1,
           W_1_0, b_1_0, sc_1_0, of_1_0,
           W_1_1, b_1_1, sc_1_1, of_1_1,
           W_2_0, b_2_0, sc_2_0, of_2_0,
           W_cls, b_cls):
    spmm = _make_spmm()

    src = edge_index[1]
    dst = edge_index[0]
    zeros = jnp.zeros((N, D), jnp.float32)

    r = lambda v: v.reshape(1, -1)

    g0 = _tc1g(feat_subg, W_0_1)

    p0a, p0b = spmm(g0, src, dst, edge_vals, zeros)

    f0 = _tc1f(feat_subg, W_0_0, r(b_0_0), r(sc_0_0), r(of_0_0))

    g1, f1 = _tc2g(p0a, p0b, r(b_0_1), r(sc_0_1), r(of_0_1), f0,
                   W_1_1[:D], W_1_1[D:])

    p1a, p1b = spmm(g1, src, dst, edge_vals, zeros)

    f0n = _tc2f(f0, f1, W_1_0[:D], W_1_0[D:], r(b_1_0), r(sc_1_0),
                r(of_1_0))

    (out,) = _tc3(p1a, p1b, r(b_1_1), r(sc_1_1), r(of_1_1), f0n,
                  W_2_0[:D], W_2_0[D:], r(b_2_0), r(sc_2_0), r(of_2_0),
                  W_cls, r(b_cls))
    return out
